# baseline (speedup 1.0000x reference)
import jax
import jax.numpy as jnp
from jax import lax
from jax.experimental import pallas as pl
from jax.experimental.pallas import tpu as pltpu
from jax.experimental.pallas import tpu_sc as plsc

_B = 4
_S = 2048
_D = 64
_N = _B * _S
_NW = 32
_BPW = _N // _NW
_WPB = _NW // _B
_OPW = _BPW // 2
_L = 16
_SP = _S // 2


def _embed_body(x_hbm, tok_hbm, pos_hbm, out_hbm, idx_v, idx2_v, rows_v, acc_v, sem):
    c = lax.axis_index("c")
    s = lax.axis_index("s")
    wid = s * 2 + c
    b = wid // _WPB
    seq0 = (wid % _WPB) * _BPW
    obase = wid * _OPW
    pbase = lax.rem(obase, _SP)

    pltpu.sync_copy(x_hbm.at[b, pl.ds(seq0, _BPW)], idx_v)

    def shift_group(g, carry):
        sl = pl.ds(g * _L, _L)
        idx2_v[sl] = lax.shift_right_logical(idx_v[sl], 1)
        return carry

    lax.fori_loop(0, _BPW // _L, shift_group, 0)

    gather = pltpu.async_copy(tok_hbm.at[idx2_v], rows_v, sem)
    pltpu.sync_copy(pos_hbm.at[pl.ds(pbase, _OPW)], acc_v)
    gather.wait()

    def add_group(g, carry):
        par = idx_v[pl.ds(g * _L, _L)] & 1
        for j in range(_L):
            off = par[j] * _D
            r = g * _L + j
            k = g * (_L // 2) + j // 2
            h = j % 2
            for ci in range(_D // _L):
                sel = rows_v[r, pl.ds(off + ci * _L, _L)]
                plsc.addupdate(acc_v.at[k, pl.ds(h * _D + ci * _L, _L)], sel)
        return carry

    lax.fori_loop(0, _BPW // _L, add_group, 0)

    pltpu.sync_copy(acc_v, out_hbm.at[pl.ds(obase, _OPW)])


def kernel(x, tok_table, pos_table):
    xi = x.astype(jnp.int32)
    tok2 = tok_table.reshape(tok_table.shape[0] // 2, 2 * _D)
    pos2 = pos_table.reshape(_SP, 2 * _D)
    mesh = plsc.VectorSubcoreMesh(core_axis_name="c", subcore_axis_name="s")
    out = pl.kernel(
        _embed_body,
        mesh=mesh,
        out_type=jax.ShapeDtypeStruct((_N // 2, 2 * _D), jnp.float32),
        scratch_types=[
            pltpu.VMEM((_BPW,), jnp.int32),
            pltpu.VMEM((_BPW,), jnp.int32),
            pltpu.VMEM((_BPW, 2 * _D), jnp.float32),
            pltpu.VMEM((_OPW, 2 * _D), jnp.float32),
            pltpu.SemaphoreType.DMA,
        ],
    )(xi, tok2, pos2)
    return out.reshape(_B, _S, _D)


# speedup vs baseline: 1.7546x; 1.7546x over previous
import jax
import jax.numpy as jnp
from jax import lax
from jax.experimental import pallas as pl
from jax.experimental.pallas import tpu as pltpu
from jax.experimental.pallas import tpu_sc as plsc

_B = 4
_S = 2048
_D = 64
_N = _B * _S
_NW = 32
_BPW = _N // _NW
_WPB = _NW // _B
_L = 16


def _embed_body(x_hbm, tok_hbm, pos_hbm, out_hbm, idx_v, rows_v, pos_v, sem):
    c = lax.axis_index("c")
    s = lax.axis_index("s")
    wid = s * 2 + c
    b = wid // _WPB
    seq0 = (wid % _WPB) * _BPW

    pltpu.sync_copy(x_hbm.at[b, pl.ds(seq0, _BPW)], idx_v)

    def fire_group(g, carry):
        iv = idx_v[pl.ds(g * _L, _L)]
        for j in range(_L):
            pltpu.async_copy(
                tok_hbm.at[iv[j] // 8, iv[j] & 7],
                rows_v.at[g * _L + j],
                sem,
            )
        return carry

    lax.fori_loop(0, _BPW // _L, fire_group, 0)

    pltpu.sync_copy(pos_hbm.at[pl.ds(seq0, _BPW)], pos_v)
    pltpu.make_async_copy(pos_hbm.at[pl.ds(0, _BPW)], rows_v, sem).wait()

    def add_row(r, carry):
        for ci in range(_D // _L):
            sl = pl.ds(ci * _L, _L)
            rows_v[r, sl] = rows_v[r, sl] + pos_v[r, sl]
        return carry

    lax.fori_loop(0, _BPW, add_row, 0)

    pltpu.sync_copy(rows_v, out_hbm.at[b, pl.ds(seq0, _BPW)])


def kernel(x, tok_table, pos_table):
    xi = x.astype(jnp.int32)
    tok3 = tok_table.reshape(tok_table.shape[0] // 8, 8, _D)
    mesh = plsc.VectorSubcoreMesh(core_axis_name="c", subcore_axis_name="s")
    out = pl.kernel(
        _embed_body,
        mesh=mesh,
        out_type=jax.ShapeDtypeStruct((_B, _S, _D), jnp.float32),
        scratch_types=[
            pltpu.VMEM((_BPW,), jnp.int32),
            pltpu.VMEM((_BPW, _D), jnp.float32),
            pltpu.VMEM((_BPW, _D), jnp.float32),
            pltpu.SemaphoreType.DMA,
        ],
    )(xi, tok3, pos_table)
    return out
